# baseline (speedup 1.0000x reference)
import functools
import numpy as np
import jax
import jax.numpy as jnp
from jax import lax
from jax.experimental import pallas as pl
from jax.experimental.pallas import tpu as pltpu
from jax.experimental.pallas import tpu_sc as plsc

_SMOOTHING = 0.1
_CONF = 1.0 - _SMOOTHING
_NB = 32

_NC, _NS, _L = 2, 16, 16
_NW = _NC * _NS


def _tc_body(x_ref, t_ref, acc_ref, *, V, fill, conf, C):
    i = pl.program_id(0)
    xb = x_ref[...]
    t = t_ref[...]
    mf = (t != 0).astype(jnp.float32)
    col = lax.broadcasted_iota(jnp.int32, xb.shape, 1)
    xt = jnp.sum(jnp.where((col == t) & (t != 0), xb, 0.0))
    rs = jnp.sum(xb, axis=1, keepdims=True)
    part = (-fill * jnp.sum(rs * mf)
            + (fill - conf) * xt
            + fill * jnp.sum(mf * xb[:, 0:1])
            + C * jnp.sum(mf))

    @pl.when(i == 0)
    def _init():
        acc_ref[...] = jnp.zeros_like(acc_ref)

    acc_ref[...] += part.reshape(1, 1)


def _tc_sum(x, t2d):
    N, V = x.shape
    fill = _SMOOTHING / (V - 2)
    C = float(fill * np.log(fill) * (V - 2) + _CONF * np.log(_CONF))
    body = functools.partial(_tc_body, V=V, fill=fill, conf=_CONF, C=C)
    return pl.pallas_call(
        body,
        grid=(N // _NB,),
        in_specs=[
            pl.BlockSpec((_NB, V), lambda i: (i, 0)),
            pl.BlockSpec((_NB, 1), lambda i: (i, 0)),
        ],
        out_specs=pl.BlockSpec((1, 1), lambda i: (0, 0)),
        out_shape=jax.ShapeDtypeStruct((1, 1), jnp.float32),
    )(x, t2d)


def kernel(x, target):
    N, V = x.shape
    tgt = target.astype(jnp.int32)
    acc = _tc_sum(x, tgt.reshape(N, 1))
    return acc[0, 0]


# speedup vs baseline: 1.0718x; 1.0718x over previous
import functools
import numpy as np
import jax
import jax.numpy as jnp
from jax import lax
from jax.experimental import pallas as pl
from jax.experimental.pallas import tpu as pltpu
from jax.experimental.pallas import tpu_sc as plsc

_SMOOTHING = 0.1
_CONF = 1.0 - _SMOOTHING
_VB = 4096
_NSPLIT = 4

_NC, _NS, _L = 2, 16, 16
_NW = _NC * _NS


def _tc_body(*refs, V, Vb, fill, conf, C, nj):
    x_refs = refs[:_NSPLIT]
    t_refs = refs[_NSPLIT:2 * _NSPLIT]
    acc_ref = refs[2 * _NSPLIT]
    j = pl.program_id(0)

    @pl.when(j == 0)
    def _init():
        acc_ref[...] = jnp.zeros_like(acc_ref)

    col = j * Vb + lax.broadcasted_iota(
        jnp.int32, x_refs[0].shape, 1)
    part = jnp.float32(0.0)
    for x_ref, t_ref in zip(x_refs, t_refs):
        xb = x_ref[...]
        t = t_ref[...]
        mf = (t != 0).astype(jnp.float32)
        xt = jnp.sum(jnp.where((col == t) & (t != 0), xb, 0.0))
        part += (fill - conf) * xt

        @pl.when(j == 0)
        def _corr():
            c0 = fill * jnp.sum(mf * xb[:, 0:1]) + C * jnp.sum(mf)
            acc_ref[...] += c0.reshape(1, 1)

        @pl.when(j < nj - 1)
        def _main():
            rs = jnp.sum(xb, axis=1, keepdims=True)
            acc_ref[...] += (-fill * jnp.sum(rs * mf)).reshape(1, 1)

        @pl.when(j == nj - 1)
        def _tail():
            rs = jnp.sum(jnp.where(col < V, xb, 0.0), axis=1, keepdims=True)
            acc_ref[...] += (-fill * jnp.sum(rs * mf)).reshape(1, 1)

    acc_ref[...] += part.reshape(1, 1)


def _tc_sum(x, t2d):
    N, V = x.shape
    nb = N // _NSPLIT
    fill = _SMOOTHING / (V - 2)
    C = float(fill * np.log(fill) * (V - 2) + _CONF * np.log(_CONF))
    nj = (V + _VB - 1) // _VB
    body = functools.partial(
        _tc_body, V=V, Vb=_VB, fill=fill, conf=_CONF, C=C, nj=nj)

    def x_spec(k):
        return pl.BlockSpec((nb, _VB), lambda j, _k=k: (_k, j))

    def t_spec(k):
        return pl.BlockSpec((nb, 1), lambda j, _k=k: (_k, 0))

    out = pl.pallas_call(
        body,
        grid=(nj,),
        in_specs=[x_spec(k) for k in range(_NSPLIT)]
        + [t_spec(k) for k in range(_NSPLIT)],
        out_specs=pl.BlockSpec((1, 1), lambda j: (0, 0)),
        out_shape=jax.ShapeDtypeStruct((1, 1), jnp.float32),
    )(*([x] * _NSPLIT + [t2d] * _NSPLIT))
    return out


def kernel(x, target):
    N, V = x.shape
    tgt = target.astype(jnp.int32)
    acc = _tc_sum(x, tgt.reshape(N, 1))
    return acc[0, 0]


# speedup vs baseline: 1.0992x; 1.0256x over previous
import functools
import numpy as np
import jax
import jax.numpy as jnp
from jax import lax
from jax.experimental import pallas as pl
from jax.experimental.pallas import tpu as pltpu
from jax.experimental.pallas import tpu_sc as plsc

_SMOOTHING = 0.1
_CONF = 1.0 - _SMOOTHING
_VB = 4096

_NC, _NS, _L = 2, 16, 16
_NW = _NC * _NS
_NSC = 512
_UN = 10


def _chunks(V):
    step = 16 * _UN
    ch = 25600
    out = []
    off = 0
    while off < V:
        ln = min(ch, V - off)
        assert ln % step == 0, (off, ln)
        out.append((off, ln))
        off += ln
    return out


def _tc_body(x_ref, t_ref, acc_ref, *, V, Vb, fill, conf, C, nj):
    j = pl.program_id(0)
    xb = x_ref[...]
    t = t_ref[...]
    mf = (t != 0).astype(jnp.float32)
    col = j * Vb + lax.broadcasted_iota(jnp.int32, xb.shape, 1)
    xt = jnp.sum(jnp.where((col == t) & (t != 0), xb, 0.0))

    @pl.when(j == 0)
    def _init():
        corr = fill * jnp.sum(mf * xb[:, 0:1]) + C * jnp.sum(mf)
        acc_ref[...] = corr.reshape(1, 1)

    @pl.when(j < nj - 1)
    def _main():
        rs = jnp.sum(xb, axis=1, keepdims=True)
        part = -fill * jnp.sum(rs * mf) + (fill - conf) * xt
        acc_ref[...] += part.reshape(1, 1)

    @pl.when(j == nj - 1)
    def _tail():
        rs = jnp.sum(jnp.where(col < V, xb, 0.0), axis=1, keepdims=True)
        part = -fill * jnp.sum(rs * mf) + (fill - conf) * xt
        acc_ref[...] += part.reshape(1, 1)


def _tc_sum(x, t2d, ntc):
    N, V = x.shape
    fill = _SMOOTHING / (V - 2)
    C = float(fill * np.log(fill) * (V - 2) + _CONF * np.log(_CONF))
    nj = (V + _VB - 1) // _VB
    body = functools.partial(
        _tc_body, V=V, Vb=_VB, fill=fill, conf=_CONF, C=C, nj=nj)
    return pl.pallas_call(
        body,
        grid=(nj,),
        in_specs=[
            pl.BlockSpec((ntc, _VB), lambda j: (0, j)),
            pl.BlockSpec((ntc, 1), lambda j: (0, 0)),
        ],
        out_specs=pl.BlockSpec((1, 1), lambda j: (0, 0)),
        out_shape=jax.ShapeDtypeStruct((1, 1), jnp.float32),
    )(x, t2d)


def _sc_rows(x, tgt, base0, nsc):
    N, V = x.shape
    fill = _SMOOTHING / (V - 2)
    C = float(fill * np.log(fill) * (V - 2) + _CONF * np.log(_CONF))
    rpw = nsc // _NW
    chunks = _chunks(V)
    ncb = len(chunks)
    mesh = plsc.VectorSubcoreMesh(core_axis_name="c", subcore_axis_name="s")

    assert rpw == _L, "one lane per row within each worker"

    @functools.partial(
        pl.kernel,
        out_type=jax.ShapeDtypeStruct((_NW, _L), jnp.float32),
        mesh=mesh,
        scratch_types=[
            pltpu.VMEM((rpw,), jnp.int32),
            pltpu.VMEM((rpw * _L,), jnp.float32),
            pltpu.VMEM((_L,), jnp.float32),
        ]
        + [pltpu.VMEM((ln,), jnp.float32) for _, ln in chunks]
        + [pltpu.SemaphoreType.DMA] * ncb,
        compiler_params=pltpu.CompilerParams(needs_layout_passes=False),
    )
    def sc_kern(x_hbm, t_hbm, out_hbm, t_v, st_v, ps_v, *bufs_sems):
        bufs = bufs_sems[:ncb]
        sems = bufs_sems[ncb:ncb + ncb]
        wid = lax.axis_index("s") * _NC + lax.axis_index("c")
        base = base0 + wid * rpw
        pltpu.sync_copy(t_hbm.at[pl.ds(base, rpw)], t_v)
        lanes = lax.iota(jnp.int32, _L)
        tvec = t_v[...]
        zeros_i = jnp.zeros((_L,), jnp.int32)

        def seg_sum(buf, ln, acc):
            def body(ii, a):
                b = ii * (16 * _UN)
                for u in range(_UN):
                    a = a + buf[pl.ds(b + u * 16, 16)]
                return a
            return lax.fori_loop(0, ln // (16 * _UN), body, acc)

        cps = {}
        for c, (off, ln) in enumerate(chunks):
            cps[(0, c)] = pltpu.async_copy(
                x_hbm.at[base, pl.ds(off, ln)], bufs[c], sems[c])

        x016 = jnp.zeros((_L,), jnp.float32)
        xt16 = jnp.zeros((_L,), jnp.float32)
        for r in range(rpw):
            acc = jnp.zeros((_L,), jnp.float32)
            for c, (off, ln) in enumerate(chunks):
                cps[(r, c)].wait()
                if c == 0:
                    cand0 = plsc.load_gather(bufs[0], [zeros_i])
                    x016 = jnp.where(lanes == r, cand0, x016)
                d = jnp.minimum(jnp.maximum(tvec - off, 0), ln - 1)
                cand = plsc.load_gather(bufs[c], [d])
                hit = (lanes == r) & (tvec >= off) & (tvec < off + ln)
                xt16 = jnp.where(hit, cand, xt16)
                acc = seg_sum(bufs[c], ln, acc)
                if r + 1 < rpw:
                    cps[(r + 1, c)] = pltpu.async_copy(
                        x_hbm.at[base + r + 1, pl.ds(off, ln)],
                        bufs[c], sems[c])
            st_v[pl.ds(r * _L, _L)] = acc

        rs16 = jnp.zeros((_L,), jnp.float32)
        for jcol in range(_L):
            rs16 = rs16 + plsc.load_gather(st_v, [lanes * _L + jcol])
        m16 = jnp.where(tvec != 0, 1.0, 0.0)
        contrib = m16 * (C - fill * (rs16 - x016 - xt16) - _CONF * xt16)
        ps_v[...] = contrib
        pltpu.sync_copy(ps_v, out_hbm.at[wid])

    return sc_kern(x, tgt)


def kernel(x, target):
    N, V = x.shape
    ntc = N - _NSC
    tgt = target.astype(jnp.int32)
    sc_parts = _sc_rows(x, tgt, ntc, _NSC)
    acc = _tc_sum(x, tgt.reshape(N, 1), ntc)
    return acc[0, 0] + jnp.sum(sc_parts)
